# baseline (speedup 1.0000x reference)
import functools

import jax
import jax.numpy as jnp
from jax.experimental import pallas as pl
from jax.experimental.pallas import tpu as pltpu

_CONF = 0.2
_IOU = 0.6
_MAXDET = 300
_MAXWH = 4096.0
_SELW = 512


def _decode_body(params_ref, preds_ref, x1_ref, y1_ref, x2_ref, y2_ref,
                 cf_ref, cl_ref, *, nc, H, W):
    l = pl.program_id(0)
    a = pl.program_id(2)
    na = pl.num_programs(2)
    h = l * na + a
    aw = params_ref[h, 0]
    ah = params_ref[h, 1]
    sw = params_ref[h, 2]
    sh = params_ref[h, 3]

    p = preds_ref[0, 0, 0]
    s = jax.nn.sigmoid(p)

    ni = jax.lax.broadcasted_iota(jnp.int32, (1, H * W), 1)
    xg = (ni % W).astype(jnp.float32)
    yg = (ni // W).astype(jnp.float32)

    xc = (s[0:1, :] * 3.0 - 1.0 + xg) * sw
    yc = (s[1:2, :] * 3.0 - 1.0 + yg) * sh
    w = ((s[2:3, :] * 2.0) ** 2 * aw) * sw
    hh = ((s[3:4, :] * 2.0) ** 2 * ah) * sh

    x1_ref[0, 0] = xc - w / 2.0
    y1_ref[0, 0] = yc - hh / 2.0
    x2_ref[0, 0] = xc + w / 2.0
    y2_ref[0, 0] = yc + hh / 2.0

    obj = s[4:5, :]
    cls_s = s[5:, :] * obj
    conf = jnp.max(cls_s, axis=0, keepdims=True)
    kio = jax.lax.broadcasted_iota(jnp.int32, (nc - 5, H * W), 0)
    cls_i = jnp.min(jnp.where(cls_s == conf, kio, jnp.int32(1 << 30)),
                    axis=0, keepdims=True)
    cf_ref[0, 0] = jnp.where(conf > _CONF, conf, 0.0)
    cl_ref[0, 0] = cls_i.astype(jnp.float32)


_GRP = 48


def _nms_body(x1_ref, y1_ref, x2_ref, y2_ref, cf_ref, cl_ref, out_ref,
              vals_scr, gmax_scr, sel_scr, nms_scr, sn_scr, sb_scr,
              *, rows, lw):
    ngrp = rows // _GRP
    grows = _GRP * lw
    vals_scr[...] = cf_ref[0]
    for g in range(8):
        if g < ngrp:
            gmax_scr[g:g + 1, :] = jnp.max(
                cf_ref[0, g * _GRP:(g + 1) * _GRP, :], axis=0, keepdims=True)
        else:
            gmax_scr[g:g + 1, :] = jnp.full((1, lw), -1.0, jnp.float32)
    sel_scr[...] = jnp.zeros((8, _SELW), jnp.float32)

    gidx = jax.lax.broadcasted_iota(jnp.int32, (8, lw), 0)
    idx48 = (jax.lax.broadcasted_iota(jnp.int32, (_GRP, lw), 0) * lw
             + jax.lax.broadcasted_iota(jnp.int32, (_GRP, lw), 1))
    lrow48 = jax.lax.broadcasted_iota(jnp.int32, (_GRP, lw), 0)
    lane48 = jax.lax.broadcasted_iota(jnp.int32, (_GRP, lw), 1)
    lane = jax.lax.broadcasted_iota(jnp.int32, (1, lw), 1)
    lane_s = jax.lax.broadcasted_iota(jnp.int32, (1, _SELW), 1)
    big = jnp.int32(1 << 30)

    def sel_body(i, _):
        gm = gmax_scr[...]
        m = jnp.max(gm)
        g = jnp.min(jnp.where(gm == m, gidx, big))
        blk = vals_scr[pl.ds(g * _GRP, _GRP), :]
        n = jnp.min(jnp.where(blk == m, idx48, big)) + g * grows
        r = n // lw
        c = n % lw
        rl = r - g * _GRP
        row = vals_scr[pl.ds(r, 1), :]
        vals_scr[pl.ds(r, 1), :] = jnp.where(lane == c, -1.0, row)
        blkc = jnp.where((lrow48 == rl) & (lane48 == c), -1.0, blk)
        gmax_scr[pl.ds(g, 1), :] = jnp.max(blkc, axis=0, keepdims=True)
        sn_scr[i] = n
        sb_scr[i, 4] = m
        return 0

    jax.lax.fori_loop(0, _MAXDET, sel_body, 0, unroll=2)

    def gat_body(t, _):
        for u in range(4):
            i = t * 4 + u
            n = sn_scr[i]
            r = n // lw
            c = n % lw
            oh = lane == c

            def ext(ref, r=r, oh=oh):
                return jnp.sum(jnp.where(oh, ref[0, pl.ds(r, 1), :], 0.0))

            scal = (ext(x1_ref), ext(y1_ref), ext(x2_ref), ext(y2_ref),
                    sb_scr[i, 4], ext(cl_ref))
            for j in (0, 1, 2, 3, 5):
                sb_scr[i, j] = scal[j]
            ohs = lane_s == i
            for j in range(6):
                old = sel_scr[pl.ds(j, 1), :]
                sel_scr[pl.ds(j, 1), :] = jnp.where(ohs, scal[j], old)
        return 0

    jax.lax.fori_loop(0, _MAXDET // 4, gat_body, 0)

    sx1 = sel_scr[0:1, :]
    sy1 = sel_scr[1:2, :]
    sx2 = sel_scr[2:3, :]
    sy2 = sel_scr[3:4, :]
    sv = sel_scr[4:5, :]
    scl = sel_scr[5:6, :]

    off = scl * _MAXWH
    nms_scr[0:1, :] = sx1 + off
    nms_scr[1:2, :] = sy1 + off
    nms_scr[2:3, :] = sx2 + off
    nms_scr[3:4, :] = sy2 + off
    nms_scr[4:5, :] = ((sx2 + off) - (sx1 + off)) * ((sy2 + off) - (sy1 + off))
    nms_scr[5:6, :] = (sv > _CONF).astype(jnp.float32)

    def nms_step(i, _):
        offi = sb_scr[i, 5] * _MAXWH
        ox1i = sb_scr[i, 0] + offi
        oy1i = sb_scr[i, 1] + offi
        ox2i = sb_scr[i, 2] + offi
        oy2i = sb_scr[i, 3] + offi
        ai = (ox2i - ox1i) * (oy2i - oy1i)
        keep = nms_scr[5:6, :]
        ki = jnp.sum(jnp.where(lane_s == i, keep, 0.0))
        ox1 = nms_scr[0:1, :]
        oy1 = nms_scr[1:2, :]
        ox2 = nms_scr[2:3, :]
        oy2 = nms_scr[3:4, :]
        area = nms_scr[4:5, :]
        iw = jnp.maximum(jnp.minimum(ox2i, ox2) - jnp.maximum(ox1i, ox1), 0.0)
        ih = jnp.maximum(jnp.minimum(oy2i, oy2) - jnp.maximum(oy1i, oy1), 0.0)
        inter = iw * ih
        iou = inter / (ai + area - inter + 1e-9)
        suppress = ((iou > _IOU) & (lane_s > i) & (ki > 0.5)).astype(
            jnp.float32)
        nms_scr[5:6, :] = keep * (1.0 - suppress)
        return 0

    jax.lax.fori_loop(0, _MAXDET, nms_step, 0)
    kf = nms_scr[5:6, :]

    out_ref[0, 0:1, :] = sx1 * kf
    out_ref[0, 1:2, :] = sy1 * kf
    out_ref[0, 2:3, :] = sx2 * kf
    out_ref[0, 3:4, :] = sy2 * kf
    out_ref[0, 4:5, :] = sv * kf
    out_ref[0, 5:6, :] = scl * kf
    out_ref[0, 6:8, :] = jnp.zeros((2, _SELW), jnp.float32)


def kernel(preds, anchors, image_size):
    L, bs, C, H, W = preds.shape
    na = anchors.shape[1]
    nc = C // na
    HW = H * W
    NH = L * na
    N = NH * HW

    img = jnp.asarray(image_size, jnp.float32)
    sh = img / jnp.float32(H)
    sw = img / jnp.float32(W)
    aw = (anchors[..., 0] / sw).reshape(NH)
    ah = (anchors[..., 1] / sh).reshape(NH)
    params = jnp.stack(
        [aw, ah, jnp.broadcast_to(sw, (NH,)), jnp.broadcast_to(sh, (NH,))],
        axis=-1)

    p5 = preds.reshape(L, bs, na, nc, HW)

    decode = pl.pallas_call(
        functools.partial(_decode_body, nc=nc, H=H, W=W),
        grid=(L, bs, na),
        in_specs=[
            pl.BlockSpec(memory_space=pltpu.SMEM),
            pl.BlockSpec((1, 1, 1, nc, HW), lambda l, b, a: (l, b, a, 0, 0)),
        ],
        out_specs=[
            pl.BlockSpec((1, 1, 1, HW), lambda l, b, a: (b, l * na + a, 0, 0))
        ] * 6,
        out_shape=[jax.ShapeDtypeStruct((bs, NH, 1, HW), jnp.float32)] * 6,
    )
    x1, y1, x2, y2, cf, cl = decode(params, p5)

    lw = 128
    rows = N // lw
    def rsh(t):
        return t.reshape(bs, rows, lw)
    x1, y1, x2, y2, cf, cl = map(rsh, (x1, y1, x2, y2, cf, cl))

    nms = pl.pallas_call(
        functools.partial(_nms_body, rows=rows, lw=lw),
        grid=(bs,),
        in_specs=[pl.BlockSpec((1, rows, lw), lambda b: (b, 0, 0))] * 6,
        out_specs=pl.BlockSpec((1, 8, _SELW), lambda b: (b, 0, 0)),
        out_shape=jax.ShapeDtypeStruct((bs, 8, _SELW), jnp.float32),
        scratch_shapes=[
            pltpu.VMEM((rows, lw), jnp.float32),
            pltpu.VMEM((8, lw), jnp.float32),
            pltpu.VMEM((8, _SELW), jnp.float32),
            pltpu.VMEM((8, _SELW), jnp.float32),
            pltpu.SMEM((_MAXDET + 4,), jnp.int32),
            pltpu.SMEM((_MAXDET + 4, 8), jnp.float32),
        ],
    )
    det8 = nms(x1, y1, x2, y2, cf, cl)
    return det8[:, :6, :_MAXDET].transpose(0, 2, 1)


# speedup vs baseline: 1.0078x; 1.0078x over previous
import functools

import jax
import jax.numpy as jnp
from jax.experimental import pallas as pl
from jax.experimental.pallas import tpu as pltpu

_CONF = 0.2
_IOU = 0.6
_MAXDET = 300
_MAXWH = 4096.0
_SELW = 512


def _decode_body(params_ref, preds_ref, x1_ref, y1_ref, x2_ref, y2_ref,
                 cf_ref, cl_ref, *, nc, H, W):
    l = pl.program_id(0)
    a = pl.program_id(2)
    na = pl.num_programs(2)
    h = l * na + a
    aw = params_ref[h, 0]
    ah = params_ref[h, 1]
    sw = params_ref[h, 2]
    sh = params_ref[h, 3]

    p = preds_ref[0, 0, 0]
    s = jax.nn.sigmoid(p)

    ni = jax.lax.broadcasted_iota(jnp.int32, (1, H * W), 1)
    xg = (ni % W).astype(jnp.float32)
    yg = (ni // W).astype(jnp.float32)

    xc = (s[0:1, :] * 3.0 - 1.0 + xg) * sw
    yc = (s[1:2, :] * 3.0 - 1.0 + yg) * sh
    w = ((s[2:3, :] * 2.0) ** 2 * aw) * sw
    hh = ((s[3:4, :] * 2.0) ** 2 * ah) * sh

    x1_ref[0, 0] = xc - w / 2.0
    y1_ref[0, 0] = yc - hh / 2.0
    x2_ref[0, 0] = xc + w / 2.0
    y2_ref[0, 0] = yc + hh / 2.0

    obj = s[4:5, :]
    cls_s = s[5:, :] * obj
    conf = jnp.max(cls_s, axis=0, keepdims=True)
    kio = jax.lax.broadcasted_iota(jnp.int32, (nc - 5, H * W), 0)
    cls_i = jnp.min(jnp.where(cls_s == conf, kio, jnp.int32(1 << 30)),
                    axis=0, keepdims=True)
    cf_ref[0, 0] = jnp.where(conf > _CONF, conf, 0.0)
    cl_ref[0, 0] = cls_i.astype(jnp.float32)


_GRP = 8


def _nms_body(x1_ref, y1_ref, x2_ref, y2_ref, cf_ref, cl_ref, out_ref,
              vals_scr, gmax_scr, sel_scr, nms_scr, sn_scr, sb_scr,
              *, rows, lw):
    ngrp = rows // _GRP
    grows = _GRP * lw
    gpad = (ngrp + 7) // 8 * 8
    vals_scr[...] = cf_ref[0]
    for g in range(gpad):
        if g < ngrp:
            gmax_scr[g:g + 1, :] = jnp.max(
                cf_ref[0, g * _GRP:(g + 1) * _GRP, :], axis=0, keepdims=True)
        else:
            gmax_scr[g:g + 1, :] = jnp.full((1, lw), -1.0, jnp.float32)
    sel_scr[...] = jnp.zeros((8, _SELW), jnp.float32)

    gidx = jax.lax.broadcasted_iota(jnp.int32, (gpad, lw), 0)
    idx8 = (jax.lax.broadcasted_iota(jnp.int32, (_GRP, lw), 0) * lw
            + jax.lax.broadcasted_iota(jnp.int32, (_GRP, lw), 1))
    lrow8 = jax.lax.broadcasted_iota(jnp.int32, (_GRP, lw), 0)
    lane8 = jax.lax.broadcasted_iota(jnp.int32, (_GRP, lw), 1)
    lane = jax.lax.broadcasted_iota(jnp.int32, (1, lw), 1)
    lane_s = jax.lax.broadcasted_iota(jnp.int32, (1, _SELW), 1)
    big = jnp.int32(1 << 30)

    def sel_body(i, _):
        gm = gmax_scr[...]
        m = jnp.max(gm)
        g = jnp.min(jnp.where(gm == m, gidx, big))
        blk = vals_scr[pl.ds(g * _GRP, _GRP), :]
        n = jnp.min(jnp.where(blk == m, idx8, big)) + g * grows
        r = n // lw
        c = n % lw
        rl = r - g * _GRP
        blkc = jnp.where((lrow8 == rl) & (lane8 == c), -1.0, blk)
        vals_scr[pl.ds(g * _GRP, _GRP), :] = blkc
        gmax_scr[pl.ds(g, 1), :] = jnp.max(blkc, axis=0, keepdims=True)
        sn_scr[i] = n
        sb_scr[i, 4] = m
        return 0

    jax.lax.fori_loop(0, _MAXDET, sel_body, 0)

    def gat_body(t, _):
        for u in range(4):
            i = t * 4 + u
            n = sn_scr[i]
            r = n // lw
            c = n % lw
            oh = lane == c

            def ext(ref, r=r, oh=oh):
                return jnp.sum(jnp.where(oh, ref[0, pl.ds(r, 1), :], 0.0))

            scal = (ext(x1_ref), ext(y1_ref), ext(x2_ref), ext(y2_ref),
                    sb_scr[i, 4], ext(cl_ref))
            for j in (0, 1, 2, 3, 5):
                sb_scr[i, j] = scal[j]
            ohs = lane_s == i
            for j in range(6):
                old = sel_scr[pl.ds(j, 1), :]
                sel_scr[pl.ds(j, 1), :] = jnp.where(ohs, scal[j], old)
        return 0

    jax.lax.fori_loop(0, _MAXDET // 4, gat_body, 0)

    sx1 = sel_scr[0:1, :]
    sy1 = sel_scr[1:2, :]
    sx2 = sel_scr[2:3, :]
    sy2 = sel_scr[3:4, :]
    sv = sel_scr[4:5, :]
    scl = sel_scr[5:6, :]

    off = scl * _MAXWH
    nms_scr[0:1, :] = sx1 + off
    nms_scr[1:2, :] = sy1 + off
    nms_scr[2:3, :] = sx2 + off
    nms_scr[3:4, :] = sy2 + off
    nms_scr[4:5, :] = ((sx2 + off) - (sx1 + off)) * ((sy2 + off) - (sy1 + off))

    def nms_step(t, keep):
        for u in range(4):
            i = t * 4 + u
            offi = sb_scr[i, 5] * _MAXWH
            ox1i = sb_scr[i, 0] + offi
            oy1i = sb_scr[i, 1] + offi
            ox2i = sb_scr[i, 2] + offi
            oy2i = sb_scr[i, 3] + offi
            ai = (ox2i - ox1i) * (oy2i - oy1i)
            ki = jnp.sum(jnp.where(lane_s == i, keep, 0.0))
            ox1 = nms_scr[0:1, :]
            oy1 = nms_scr[1:2, :]
            ox2 = nms_scr[2:3, :]
            oy2 = nms_scr[3:4, :]
            area = nms_scr[4:5, :]
            iw = jnp.maximum(
                jnp.minimum(ox2i, ox2) - jnp.maximum(ox1i, ox1), 0.0)
            ih = jnp.maximum(
                jnp.minimum(oy2i, oy2) - jnp.maximum(oy1i, oy1), 0.0)
            inter = iw * ih
            iou = inter / (ai + area - inter + 1e-9)
            suppress = ((iou > _IOU) & (lane_s > i) & (ki > 0.5)).astype(
                jnp.float32)
            keep = keep * (1.0 - suppress)
        return keep

    kf = jax.lax.fori_loop(0, _MAXDET // 4, nms_step,
                           (sv > _CONF).astype(jnp.float32))

    out_ref[0, 0:1, :] = sx1 * kf
    out_ref[0, 1:2, :] = sy1 * kf
    out_ref[0, 2:3, :] = sx2 * kf
    out_ref[0, 3:4, :] = sy2 * kf
    out_ref[0, 4:5, :] = sv * kf
    out_ref[0, 5:6, :] = scl * kf
    out_ref[0, 6:8, :] = jnp.zeros((2, _SELW), jnp.float32)


def kernel(preds, anchors, image_size):
    L, bs, C, H, W = preds.shape
    na = anchors.shape[1]
    nc = C // na
    HW = H * W
    NH = L * na
    N = NH * HW

    img = jnp.asarray(image_size, jnp.float32)
    sh = img / jnp.float32(H)
    sw = img / jnp.float32(W)
    aw = (anchors[..., 0] / sw).reshape(NH)
    ah = (anchors[..., 1] / sh).reshape(NH)
    params = jnp.stack(
        [aw, ah, jnp.broadcast_to(sw, (NH,)), jnp.broadcast_to(sh, (NH,))],
        axis=-1)

    p5 = preds.reshape(L, bs, na, nc, HW)

    decode = pl.pallas_call(
        functools.partial(_decode_body, nc=nc, H=H, W=W),
        grid=(L, bs, na),
        in_specs=[
            pl.BlockSpec(memory_space=pltpu.SMEM),
            pl.BlockSpec((1, 1, 1, nc, HW), lambda l, b, a: (l, b, a, 0, 0)),
        ],
        out_specs=[
            pl.BlockSpec((1, 1, 1, HW), lambda l, b, a: (b, l * na + a, 0, 0))
        ] * 6,
        out_shape=[jax.ShapeDtypeStruct((bs, NH, 1, HW), jnp.float32)] * 6,
    )
    x1, y1, x2, y2, cf, cl = decode(params, p5)

    lw = 128
    rows = N // lw
    def rsh(t):
        return t.reshape(bs, rows, lw)
    x1, y1, x2, y2, cf, cl = map(rsh, (x1, y1, x2, y2, cf, cl))

    nms = pl.pallas_call(
        functools.partial(_nms_body, rows=rows, lw=lw),
        grid=(bs,),
        in_specs=[pl.BlockSpec((1, rows, lw), lambda b: (b, 0, 0))] * 6,
        out_specs=pl.BlockSpec((1, 8, _SELW), lambda b: (b, 0, 0)),
        out_shape=jax.ShapeDtypeStruct((bs, 8, _SELW), jnp.float32),
        scratch_shapes=[
            pltpu.VMEM((rows, lw), jnp.float32),
            pltpu.VMEM(((rows // _GRP + 7) // 8 * 8, lw), jnp.float32),
            pltpu.VMEM((8, _SELW), jnp.float32),
            pltpu.VMEM((8, _SELW), jnp.float32),
            pltpu.SMEM((_MAXDET + 4,), jnp.int32),
            pltpu.SMEM((_MAXDET + 4, 8), jnp.float32),
        ],
    )
    det8 = nms(x1, y1, x2, y2, cf, cl)
    return det8[:, :6, :_MAXDET].transpose(0, 2, 1)


# speedup vs baseline: 2.0887x; 2.0725x over previous
import functools

import jax
import jax.numpy as jnp
from jax.experimental import pallas as pl
from jax.experimental.pallas import tpu as pltpu

_CONF = 0.2
_IOU = 0.6
_MAXDET = 300
_MAXWH = 4096.0
_SELW = 384


def _decode_body(params_ref, preds_ref, x1_ref, y1_ref, x2_ref, y2_ref,
                 cf_ref, cl_ref, *, nc, H, W):
    l = pl.program_id(0)
    a = pl.program_id(2)
    na = pl.num_programs(2)
    h = l * na + a
    aw = params_ref[h, 0]
    ah = params_ref[h, 1]
    sw = params_ref[h, 2]
    sh = params_ref[h, 3]

    p = preds_ref[0, 0, 0]
    s = jax.nn.sigmoid(p)

    ni = jax.lax.broadcasted_iota(jnp.int32, (1, H * W), 1)
    xg = (ni % W).astype(jnp.float32)
    yg = (ni // W).astype(jnp.float32)

    xc = (s[0:1, :] * 3.0 - 1.0 + xg) * sw
    yc = (s[1:2, :] * 3.0 - 1.0 + yg) * sh
    w = ((s[2:3, :] * 2.0) ** 2 * aw) * sw
    hh = ((s[3:4, :] * 2.0) ** 2 * ah) * sh

    x1_ref[0, 0] = xc - w / 2.0
    y1_ref[0, 0] = yc - hh / 2.0
    x2_ref[0, 0] = xc + w / 2.0
    y2_ref[0, 0] = yc + hh / 2.0

    obj = s[4:5, :]
    cls_s = s[5:, :] * obj
    conf = jnp.max(cls_s, axis=0, keepdims=True)
    kio = jax.lax.broadcasted_iota(jnp.int32, (nc - 5, H * W), 0)
    cls_i = jnp.min(jnp.where(cls_s == conf, kio, jnp.int32(1 << 30)),
                    axis=0, keepdims=True)
    cf_ref[0, 0] = jnp.where(conf > _CONF, conf, 0.0)
    cl_ref[0, 0] = cls_i.astype(jnp.float32)


def _nms_body(x1_ref, y1_ref, x2_ref, y2_ref, cf_ref, cl_ref, out_ref,
              vals_scr, rh_scr, ch_scr, x6_scr, sel_scr, nms_scr, nmst_scr,
              *, rows, lw, bs):
    pad = _SELW
    refs = (x1_ref, y1_ref, x2_ref, y2_ref, cf_ref, cl_ref)
    for b in range(bs):
        vals_scr[b] = cf_ref[b]
        rh_scr[b] = jnp.zeros((pad, pad), jnp.float32)
        ch_scr[b] = jnp.zeros((pad, lw), jnp.float32)
        for f, ref in enumerate(refs):
            x6_scr[b, 0:rows, f * lw:(f + 1) * lw] = ref[b]
        x6_scr[b, rows:pad, :] = jnp.zeros((pad - rows, 6 * lw), jnp.float32)

    lane = jax.lax.broadcasted_iota(jnp.int32, (1, lw), 1)
    lane_s = jax.lax.broadcasted_iota(jnp.int32, (1, _SELW), 1)
    big = jnp.int32(1 << 30)
    nchunk = rows // 8
    iota8 = (jax.lax.broadcasted_iota(jnp.int32, (8, lw), 0) * lw
             + jax.lax.broadcasted_iota(jnp.int32, (8, lw), 1))
    cspan = 8 * lw

    def sel_body(i, _):
        for b in range(bs):
            chunks = [vals_scr[b, k * 8:(k + 1) * 8, :]
                      for k in range(nchunk)]
            lv = chunks
            while len(lv) > 1:
                nxt = [jnp.maximum(lv[2 * j], lv[2 * j + 1])
                       for j in range(len(lv) // 2)]
                if len(lv) % 2:
                    nxt.append(lv[-1])
                lv = nxt
            m = jnp.max(jnp.max(lv[0], axis=0, keepdims=True), axis=1,
                        keepdims=True)
            pos = [(jnp.where(c == m, iota8, big), 1) for c in chunks]
            while len(pos) > 1:
                nxt = []
                for j in range(len(pos) // 2):
                    (va, la), (vb, lb) = pos[2 * j], pos[2 * j + 1]
                    nxt.append((jnp.minimum(va, vb + la * cspan), la + lb))
                if len(pos) % 2:
                    nxt.append(pos[-1])
                pos = nxt
            nv = jnp.min(jnp.min(pos[0][0], axis=0, keepdims=True), axis=1,
                         keepdims=True)
            for k in range(nchunk):
                lc = nv - k * cspan
                vals_scr[b, k * 8:(k + 1) * 8, :] = jnp.where(
                    iota8 == lc, -1.0, vals_scr[b, k * 8:(k + 1) * 8, :])
            rv = nv // lw
            cv = nv - rv * lw
            rh_scr[b, pl.ds(i, 1), :] = (lane_s == rv).astype(jnp.float32)
            ch_scr[b, pl.ds(i, 1), :] = (lane == cv).astype(jnp.float32)
        return 0

    jax.lax.fori_loop(0, _MAXDET, sel_body, 0)

    ident = (jax.lax.broadcasted_iota(jnp.int32, (lw, lw), 0)
             == jax.lax.broadcasted_iota(jnp.int32, (lw, lw), 1)
             ).astype(jnp.float32)
    keeps = []
    for b in range(bs):
        rowsel6 = jax.lax.dot_general(
            rh_scr[b], x6_scr[b], (((1,), (0,)), ((), ())),
            precision=jax.lax.Precision.HIGHEST,
            preferred_element_type=jnp.float32)
        ch = ch_scr[b]
        selt = jnp.zeros((pad, lw), jnp.float32)
        for f in range(6):
            colv = jnp.sum(rowsel6[:, f * lw:(f + 1) * lw] * ch,
                           axis=1, keepdims=True)
            selt = selt + jnp.where(lane == f, colv, 0.0)
        t = jax.lax.dot_general(ident, selt, (((1,), (1,)), ((), ())),
                                precision=jax.lax.Precision.HIGHEST,
                                preferred_element_type=jnp.float32)
        for f in range(6):
            sel_scr[b, f:f + 1, :] = t[f:f + 1, :]

        sx1 = sel_scr[b, 0:1, :]
        sy1 = sel_scr[b, 1:2, :]
        sx2 = sel_scr[b, 2:3, :]
        sy2 = sel_scr[b, 3:4, :]
        sv = sel_scr[b, 4:5, :]
        scl = sel_scr[b, 5:6, :]

        off = scl * _MAXWH
        ox1 = sx1 + off
        oy1 = sy1 + off
        ox2 = sx2 + off
        oy2 = sy2 + off
        nms_scr[b, 0:1, :] = ox1
        nms_scr[b, 1:2, :] = oy1
        nms_scr[b, 2:3, :] = ox2
        nms_scr[b, 3:4, :] = oy2
        nms_scr[b, 4:5, :] = (ox2 - ox1) * (oy2 - oy1)

        def colof(f, selt=selt):
            return jnp.sum(jnp.where(lane == f, selt, 0.0), axis=1,
                           keepdims=True)

        offc = colof(5) * _MAXWH
        ox1c = colof(0) + offc
        oy1c = colof(1) + offc
        ox2c = colof(2) + offc
        oy2c = colof(3) + offc
        areac = (ox2c - ox1c) * (oy2c - oy1c)
        nmst_scr[b] = (jnp.where(lane == 0, ox1c, 0.0)
                       + jnp.where(lane == 1, oy1c, 0.0)
                       + jnp.where(lane == 2, ox2c, 0.0)
                       + jnp.where(lane == 3, oy2c, 0.0)
                       + jnp.where(lane == 4, areac, 0.0))
        keeps.append((sv > _CONF).astype(jnp.float32))

    def nms_step(t, keepT):
        keepT = list(keepT)
        for u in range(2):
            i = t * 2 + u
            for b in range(bs):
                keep = keepT[b]
                rowi = nmst_scr[b, pl.ds(i, 1), :]

                def bx5(f, rowi=rowi):
                    return jnp.sum(jnp.where(lane == f, rowi, 0.0), axis=1,
                                   keepdims=True)

                x1i = bx5(0)
                y1i = bx5(1)
                x2i = bx5(2)
                y2i = bx5(3)
                ai = bx5(4)
                ki = jnp.sum(jnp.where(lane_s == i, keep, 0.0), axis=1,
                             keepdims=True)
                iw = jnp.maximum(
                    jnp.minimum(x2i, nms_scr[b, 2:3, :])
                    - jnp.maximum(x1i, nms_scr[b, 0:1, :]), 0.0)
                ih = jnp.maximum(
                    jnp.minimum(y2i, nms_scr[b, 3:4, :])
                    - jnp.maximum(y1i, nms_scr[b, 1:2, :]), 0.0)
                inter = iw * ih
                iou = inter / (ai + nms_scr[b, 4:5, :] - inter + 1e-9)
                suppress = ((iou > _IOU) & (lane_s > i)
                            & (ki > 0.5)).astype(jnp.float32)
                keepT[b] = keep * (1.0 - suppress)
        return tuple(keepT)

    kfs = jax.lax.fori_loop(0, _MAXDET // 2, nms_step, tuple(keeps))

    for b in range(bs):
        kf = kfs[b]
        out_ref[b, 0:1, :] = sel_scr[b, 0:1, :] * kf
        out_ref[b, 1:2, :] = sel_scr[b, 1:2, :] * kf
        out_ref[b, 2:3, :] = sel_scr[b, 2:3, :] * kf
        out_ref[b, 3:4, :] = sel_scr[b, 3:4, :] * kf
        out_ref[b, 4:5, :] = sel_scr[b, 4:5, :] * kf
        out_ref[b, 5:6, :] = sel_scr[b, 5:6, :] * kf
        out_ref[b, 6:8, :] = jnp.zeros((2, _SELW), jnp.float32)


def kernel(preds, anchors, image_size):
    L, bs, C, H, W = preds.shape
    na = anchors.shape[1]
    nc = C // na
    HW = H * W
    NH = L * na
    N = NH * HW

    img = jnp.asarray(image_size, jnp.float32)
    sh = img / jnp.float32(H)
    sw = img / jnp.float32(W)
    aw = (anchors[..., 0] / sw).reshape(NH)
    ah = (anchors[..., 1] / sh).reshape(NH)
    params = jnp.stack(
        [aw, ah, jnp.broadcast_to(sw, (NH,)), jnp.broadcast_to(sh, (NH,))],
        axis=-1)

    p5 = preds.reshape(L, bs, na, nc, HW)

    decode = pl.pallas_call(
        functools.partial(_decode_body, nc=nc, H=H, W=W),
        grid=(L, bs, na),
        in_specs=[
            pl.BlockSpec(memory_space=pltpu.SMEM),
            pl.BlockSpec((1, 1, 1, nc, HW), lambda l, b, a: (l, b, a, 0, 0)),
        ],
        out_specs=[
            pl.BlockSpec((1, 1, 1, HW), lambda l, b, a: (b, l * na + a, 0, 0))
        ] * 6,
        out_shape=[jax.ShapeDtypeStruct((bs, NH, 1, HW), jnp.float32)] * 6,
    )
    x1, y1, x2, y2, cf, cl = decode(params, p5)

    lw = 128
    rows = N // lw
    def rsh(t):
        return t.reshape(bs, rows, lw)
    x1, y1, x2, y2, cf, cl = map(rsh, (x1, y1, x2, y2, cf, cl))

    nms = pl.pallas_call(
        functools.partial(_nms_body, rows=rows, lw=lw, bs=bs),
        out_shape=jax.ShapeDtypeStruct((bs, 8, _SELW), jnp.float32),
        scratch_shapes=[
            pltpu.VMEM((bs, rows, lw), jnp.float32),
            pltpu.VMEM((bs, _SELW, _SELW), jnp.float32),
            pltpu.VMEM((bs, _SELW, lw), jnp.float32),
            pltpu.VMEM((bs, _SELW, 6 * lw), jnp.float32),
            pltpu.VMEM((bs, 8, _SELW), jnp.float32),
            pltpu.VMEM((bs, 8, _SELW), jnp.float32),
            pltpu.VMEM((bs, _SELW, lw), jnp.float32),
        ],
    )
    det8 = nms(x1, y1, x2, y2, cf, cl)
    return det8[:, :6, :_MAXDET].transpose(0, 2, 1)


# speedup vs baseline: 2.1272x; 1.0185x over previous
import functools

import jax
import jax.numpy as jnp
from jax.experimental import pallas as pl
from jax.experimental.pallas import tpu as pltpu

_CONF = 0.2
_IOU = 0.6
_MAXDET = 300
_MAXWH = 4096.0
_SELW = 384


def _decode_body(params_ref, preds_ref, x1_ref, y1_ref, x2_ref, y2_ref,
                 cf_ref, cl_ref, *, nc, H, W):
    l = pl.program_id(0)
    a = pl.program_id(2)
    na = pl.num_programs(2)
    h = l * na + a
    aw = params_ref[h, 0]
    ah = params_ref[h, 1]
    sw = params_ref[h, 2]
    sh = params_ref[h, 3]

    p = preds_ref[0, 0, 0]
    s = jax.nn.sigmoid(p)

    ni = jax.lax.broadcasted_iota(jnp.int32, (1, H * W), 1)
    xg = (ni % W).astype(jnp.float32)
    yg = (ni // W).astype(jnp.float32)

    xc = (s[0:1, :] * 3.0 - 1.0 + xg) * sw
    yc = (s[1:2, :] * 3.0 - 1.0 + yg) * sh
    w = ((s[2:3, :] * 2.0) ** 2 * aw) * sw
    hh = ((s[3:4, :] * 2.0) ** 2 * ah) * sh

    x1_ref[0, 0] = xc - w / 2.0
    y1_ref[0, 0] = yc - hh / 2.0
    x2_ref[0, 0] = xc + w / 2.0
    y2_ref[0, 0] = yc + hh / 2.0

    obj = s[4:5, :]
    cls_s = s[5:, :] * obj
    conf = jnp.max(cls_s, axis=0, keepdims=True)
    kio = jax.lax.broadcasted_iota(jnp.int32, (nc - 5, H * W), 0)
    cls_i = jnp.min(jnp.where(cls_s == conf, kio, jnp.int32(1 << 30)),
                    axis=0, keepdims=True)
    cf_ref[0, 0] = jnp.where(conf > _CONF, conf, 0.0)
    cl_ref[0, 0] = cls_i.astype(jnp.float32)


def _nms_body(x1_ref, y1_ref, x2_ref, y2_ref, cf_ref, cl_ref, out_ref,
              vals_scr, rh_scr, ch_scr, x6_scr, sel_scr, nms_scr, nmst_scr,
              *, rows, lw, bs):
    pad = _SELW
    refs = (x1_ref, y1_ref, x2_ref, y2_ref, cf_ref, cl_ref)
    for b in range(bs):
        vals_scr[b] = cf_ref[b]
        rh_scr[b] = jnp.zeros((pad, pad), jnp.float32)
        ch_scr[b] = jnp.zeros((pad, lw), jnp.float32)
        for f, ref in enumerate(refs):
            x6_scr[b, 0:rows, f * lw:(f + 1) * lw] = ref[b]
        x6_scr[b, rows:pad, :] = jnp.zeros((pad - rows, 6 * lw), jnp.float32)

    lane = jax.lax.broadcasted_iota(jnp.int32, (1, lw), 1)
    lane_s = jax.lax.broadcasted_iota(jnp.int32, (1, _SELW), 1)
    big = jnp.int32(1 << 30)
    nchunk = rows // 8
    iota8 = (jax.lax.broadcasted_iota(jnp.int32, (8, lw), 0) * lw
             + jax.lax.broadcasted_iota(jnp.int32, (8, lw), 1))
    cspan = 8 * lw

    def sel_body(i, _):
        for b in range(bs):
            chunks = [vals_scr[b, k * 8:(k + 1) * 8, :]
                      for k in range(nchunk)]
            lv = chunks
            while len(lv) > 1:
                nxt = [jnp.maximum(lv[2 * j], lv[2 * j + 1])
                       for j in range(len(lv) // 2)]
                if len(lv) % 2:
                    nxt.append(lv[-1])
                lv = nxt
            m = jnp.max(jnp.max(lv[0], axis=0, keepdims=True), axis=1,
                        keepdims=True)
            pos = [(jnp.where(c == m, iota8, big), 1) for c in chunks]
            while len(pos) > 1:
                nxt = []
                for j in range(len(pos) // 2):
                    (va, la), (vb, lb) = pos[2 * j], pos[2 * j + 1]
                    nxt.append((jnp.minimum(va, vb + la * cspan), la + lb))
                if len(pos) % 2:
                    nxt.append(pos[-1])
                pos = nxt
            nv = jnp.min(jnp.min(pos[0][0], axis=0, keepdims=True), axis=1,
                         keepdims=True)
            for k in range(nchunk):
                lc = nv - k * cspan
                vals_scr[b, k * 8:(k + 1) * 8, :] = jnp.where(
                    iota8 == lc, -1.0, vals_scr[b, k * 8:(k + 1) * 8, :])
            rv = nv // lw
            cv = nv - rv * lw
            rh_scr[b, pl.ds(i, 1), :] = (lane_s == rv).astype(jnp.float32)
            ch_scr[b, pl.ds(i, 1), :] = (lane == cv).astype(jnp.float32)
        return 0

    jax.lax.fori_loop(0, _MAXDET, sel_body, 0, unroll=2)

    ident = (jax.lax.broadcasted_iota(jnp.int32, (lw, lw), 0)
             == jax.lax.broadcasted_iota(jnp.int32, (lw, lw), 1)
             ).astype(jnp.float32)
    keeps = []
    for b in range(bs):
        rowsel6 = jax.lax.dot_general(
            rh_scr[b], x6_scr[b], (((1,), (0,)), ((), ())),
            precision=jax.lax.Precision.HIGHEST,
            preferred_element_type=jnp.float32)
        ch = ch_scr[b]
        selt = jnp.zeros((pad, lw), jnp.float32)
        for f in range(6):
            colv = jnp.sum(rowsel6[:, f * lw:(f + 1) * lw] * ch,
                           axis=1, keepdims=True)
            selt = selt + jnp.where(lane == f, colv, 0.0)
        t = jax.lax.dot_general(ident, selt, (((1,), (1,)), ((), ())),
                                precision=jax.lax.Precision.HIGHEST,
                                preferred_element_type=jnp.float32)
        for f in range(6):
            sel_scr[b, f:f + 1, :] = t[f:f + 1, :]

        sx1 = sel_scr[b, 0:1, :]
        sy1 = sel_scr[b, 1:2, :]
        sx2 = sel_scr[b, 2:3, :]
        sy2 = sel_scr[b, 3:4, :]
        sv = sel_scr[b, 4:5, :]
        scl = sel_scr[b, 5:6, :]

        off = scl * _MAXWH
        ox1 = sx1 + off
        oy1 = sy1 + off
        ox2 = sx2 + off
        oy2 = sy2 + off
        nms_scr[b, 0:1, :] = ox1
        nms_scr[b, 1:2, :] = oy1
        nms_scr[b, 2:3, :] = ox2
        nms_scr[b, 3:4, :] = oy2
        nms_scr[b, 4:5, :] = (ox2 - ox1) * (oy2 - oy1)

        def colof(f, selt=selt):
            return jnp.sum(jnp.where(lane == f, selt, 0.0), axis=1,
                           keepdims=True)

        offc = colof(5) * _MAXWH
        ox1c = colof(0) + offc
        oy1c = colof(1) + offc
        ox2c = colof(2) + offc
        oy2c = colof(3) + offc
        areac = (ox2c - ox1c) * (oy2c - oy1c)
        nmst_scr[b] = (jnp.where(lane == 0, ox1c, 0.0)
                       + jnp.where(lane == 1, oy1c, 0.0)
                       + jnp.where(lane == 2, ox2c, 0.0)
                       + jnp.where(lane == 3, oy2c, 0.0)
                       + jnp.where(lane == 4, areac, 0.0))
        keeps.append((sv > _CONF).astype(jnp.float32))

    def nms_step(t, keepT):
        keepT = list(keepT)
        for u in range(2):
            i = t * 2 + u
            for b in range(bs):
                keep = keepT[b]
                rowi = nmst_scr[b, pl.ds(i, 1), :]

                def bx5(f, rowi=rowi):
                    return jnp.sum(jnp.where(lane == f, rowi, 0.0), axis=1,
                                   keepdims=True)

                x1i = bx5(0)
                y1i = bx5(1)
                x2i = bx5(2)
                y2i = bx5(3)
                ai = bx5(4)
                ki = jnp.sum(jnp.where(lane_s == i, keep, 0.0), axis=1,
                             keepdims=True)
                iw = jnp.maximum(
                    jnp.minimum(x2i, nms_scr[b, 2:3, :])
                    - jnp.maximum(x1i, nms_scr[b, 0:1, :]), 0.0)
                ih = jnp.maximum(
                    jnp.minimum(y2i, nms_scr[b, 3:4, :])
                    - jnp.maximum(y1i, nms_scr[b, 1:2, :]), 0.0)
                inter = iw * ih
                iou = inter / (ai + nms_scr[b, 4:5, :] - inter + 1e-9)
                suppress = ((iou > _IOU) & (lane_s > i)
                            & (ki > 0.5)).astype(jnp.float32)
                keepT[b] = keep * (1.0 - suppress)
        return tuple(keepT)

    kfs = jax.lax.fori_loop(0, _MAXDET // 2, nms_step, tuple(keeps))

    for b in range(bs):
        kf = kfs[b]
        out_ref[b, 0:1, :] = sel_scr[b, 0:1, :] * kf
        out_ref[b, 1:2, :] = sel_scr[b, 1:2, :] * kf
        out_ref[b, 2:3, :] = sel_scr[b, 2:3, :] * kf
        out_ref[b, 3:4, :] = sel_scr[b, 3:4, :] * kf
        out_ref[b, 4:5, :] = sel_scr[b, 4:5, :] * kf
        out_ref[b, 5:6, :] = sel_scr[b, 5:6, :] * kf
        out_ref[b, 6:8, :] = jnp.zeros((2, _SELW), jnp.float32)


def kernel(preds, anchors, image_size):
    L, bs, C, H, W = preds.shape
    na = anchors.shape[1]
    nc = C // na
    HW = H * W
    NH = L * na
    N = NH * HW

    img = jnp.asarray(image_size, jnp.float32)
    sh = img / jnp.float32(H)
    sw = img / jnp.float32(W)
    aw = (anchors[..., 0] / sw).reshape(NH)
    ah = (anchors[..., 1] / sh).reshape(NH)
    params = jnp.stack(
        [aw, ah, jnp.broadcast_to(sw, (NH,)), jnp.broadcast_to(sh, (NH,))],
        axis=-1)

    p5 = preds.reshape(L, bs, na, nc, HW)

    decode = pl.pallas_call(
        functools.partial(_decode_body, nc=nc, H=H, W=W),
        grid=(L, bs, na),
        in_specs=[
            pl.BlockSpec(memory_space=pltpu.SMEM),
            pl.BlockSpec((1, 1, 1, nc, HW), lambda l, b, a: (l, b, a, 0, 0)),
        ],
        out_specs=[
            pl.BlockSpec((1, 1, 1, HW), lambda l, b, a: (b, l * na + a, 0, 0))
        ] * 6,
        out_shape=[jax.ShapeDtypeStruct((bs, NH, 1, HW), jnp.float32)] * 6,
    )
    x1, y1, x2, y2, cf, cl = decode(params, p5)

    lw = 128
    rows = N // lw
    def rsh(t):
        return t.reshape(bs, rows, lw)
    x1, y1, x2, y2, cf, cl = map(rsh, (x1, y1, x2, y2, cf, cl))

    nms = pl.pallas_call(
        functools.partial(_nms_body, rows=rows, lw=lw, bs=bs),
        out_shape=jax.ShapeDtypeStruct((bs, 8, _SELW), jnp.float32),
        scratch_shapes=[
            pltpu.VMEM((bs, rows, lw), jnp.float32),
            pltpu.VMEM((bs, _SELW, _SELW), jnp.float32),
            pltpu.VMEM((bs, _SELW, lw), jnp.float32),
            pltpu.VMEM((bs, _SELW, 6 * lw), jnp.float32),
            pltpu.VMEM((bs, 8, _SELW), jnp.float32),
            pltpu.VMEM((bs, 8, _SELW), jnp.float32),
            pltpu.VMEM((bs, _SELW, lw), jnp.float32),
        ],
    )
    det8 = nms(x1, y1, x2, y2, cf, cl)
    return det8[:, :6, :_MAXDET].transpose(0, 2, 1)


# speedup vs baseline: 2.1702x; 1.0202x over previous
import functools

import jax
import jax.numpy as jnp
from jax.experimental import pallas as pl
from jax.experimental.pallas import tpu as pltpu

_CONF = 0.2
_IOU = 0.6
_MAXDET = 300
_MAXWH = 4096.0
_SELW = 384


def _decode_body(params_ref, preds_ref, x1_ref, y1_ref, x2_ref, y2_ref,
                 cf_ref, cl_ref, *, nc, H, W):
    l = pl.program_id(0)
    a = pl.program_id(2)
    na = pl.num_programs(2)
    h = l * na + a
    aw = params_ref[h, 0]
    ah = params_ref[h, 1]
    sw = params_ref[h, 2]
    sh = params_ref[h, 3]

    p = preds_ref[0, 0, 0]
    s = jax.nn.sigmoid(p)

    ni = jax.lax.broadcasted_iota(jnp.int32, (1, H * W), 1)
    xg = (ni % W).astype(jnp.float32)
    yg = (ni // W).astype(jnp.float32)

    xc = (s[0:1, :] * 3.0 - 1.0 + xg) * sw
    yc = (s[1:2, :] * 3.0 - 1.0 + yg) * sh
    w = ((s[2:3, :] * 2.0) ** 2 * aw) * sw
    hh = ((s[3:4, :] * 2.0) ** 2 * ah) * sh

    x1_ref[0, 0] = xc - w / 2.0
    y1_ref[0, 0] = yc - hh / 2.0
    x2_ref[0, 0] = xc + w / 2.0
    y2_ref[0, 0] = yc + hh / 2.0

    obj = s[4:5, :]
    cls_s = s[5:, :] * obj
    conf = jnp.max(cls_s, axis=0, keepdims=True)
    kio = jax.lax.broadcasted_iota(jnp.int32, (nc - 5, H * W), 0)
    cls_i = jnp.min(jnp.where(cls_s == conf, kio, jnp.int32(1 << 30)),
                    axis=0, keepdims=True)
    cf_ref[0, 0] = jnp.where(conf > _CONF, conf, 0.0)
    cl_ref[0, 0] = cls_i.astype(jnp.float32)


def _nms_body(x1_ref, y1_ref, x2_ref, y2_ref, cf_ref, cl_ref, out_ref,
              vals_scr, rh_scr, ch_scr, x6_scr, sel_scr, nms_scr, nmst_scr,
              *, rows, lw, bs):
    pad = _SELW
    refs = (x1_ref, y1_ref, x2_ref, y2_ref, cf_ref, cl_ref)
    for b in range(bs):
        vals_scr[b] = cf_ref[b]
        rh_scr[b] = jnp.zeros((pad, pad), jnp.float32)
        ch_scr[b] = jnp.zeros((pad, lw), jnp.float32)
        for f, ref in enumerate(refs):
            x6_scr[b, 0:rows, f * lw:(f + 1) * lw] = ref[b]
        x6_scr[b, rows:pad, :] = jnp.zeros((pad - rows, 6 * lw), jnp.float32)

    lane = jax.lax.broadcasted_iota(jnp.int32, (1, lw), 1)
    lane_s = jax.lax.broadcasted_iota(jnp.int32, (1, _SELW), 1)
    big = jnp.int32(1 << 30)
    nchunk = rows // 8
    iota8 = (jax.lax.broadcasted_iota(jnp.int32, (8, lw), 0) * lw
             + jax.lax.broadcasted_iota(jnp.int32, (8, lw), 1))
    cspan = 8 * lw

    def sel_body(i, _):
        for b in range(bs):
            chunks = [vals_scr[b, k * 8:(k + 1) * 8, :]
                      for k in range(nchunk)]
            lv = chunks
            while len(lv) > 1:
                nxt = [jnp.maximum(lv[2 * j], lv[2 * j + 1])
                       for j in range(len(lv) // 2)]
                if len(lv) % 2:
                    nxt.append(lv[-1])
                lv = nxt
            m = jnp.max(jnp.max(lv[0], axis=0, keepdims=True), axis=1,
                        keepdims=True)
            pos = [(jnp.where(c == m, iota8, big), 1) for c in chunks]
            while len(pos) > 1:
                nxt = []
                for j in range(len(pos) // 2):
                    (va, la), (vb, lb) = pos[2 * j], pos[2 * j + 1]
                    nxt.append((jnp.minimum(va, vb + la * cspan), la + lb))
                if len(pos) % 2:
                    nxt.append(pos[-1])
                pos = nxt
            nv = jnp.min(jnp.min(pos[0][0], axis=0, keepdims=True), axis=1,
                         keepdims=True)
            for k in range(nchunk):
                lc = nv - k * cspan
                vals_scr[b, k * 8:(k + 1) * 8, :] = jnp.where(
                    iota8 == lc, -1.0, vals_scr[b, k * 8:(k + 1) * 8, :])
            rv = nv // lw
            cv = nv - rv * lw
            rh_scr[b, pl.ds(i, 1), :] = (lane_s == rv).astype(jnp.float32)
            ch_scr[b, pl.ds(i, 1), :] = (lane == cv).astype(jnp.float32)
        return 0

    jax.lax.fori_loop(0, _MAXDET, sel_body, 0, unroll=4)

    ident = (jax.lax.broadcasted_iota(jnp.int32, (lw, lw), 0)
             == jax.lax.broadcasted_iota(jnp.int32, (lw, lw), 1)
             ).astype(jnp.float32)
    keeps = []
    for b in range(bs):
        rowsel6 = jax.lax.dot_general(
            rh_scr[b], x6_scr[b], (((1,), (0,)), ((), ())),
            precision=jax.lax.Precision.HIGHEST,
            preferred_element_type=jnp.float32)
        ch = ch_scr[b]
        selt = jnp.zeros((pad, lw), jnp.float32)
        for f in range(6):
            colv = jnp.sum(rowsel6[:, f * lw:(f + 1) * lw] * ch,
                           axis=1, keepdims=True)
            selt = selt + jnp.where(lane == f, colv, 0.0)
        t = jax.lax.dot_general(ident, selt, (((1,), (1,)), ((), ())),
                                precision=jax.lax.Precision.HIGHEST,
                                preferred_element_type=jnp.float32)
        for f in range(6):
            sel_scr[b, f:f + 1, :] = t[f:f + 1, :]

        sx1 = sel_scr[b, 0:1, :]
        sy1 = sel_scr[b, 1:2, :]
        sx2 = sel_scr[b, 2:3, :]
        sy2 = sel_scr[b, 3:4, :]
        sv = sel_scr[b, 4:5, :]
        scl = sel_scr[b, 5:6, :]

        off = scl * _MAXWH
        ox1 = sx1 + off
        oy1 = sy1 + off
        ox2 = sx2 + off
        oy2 = sy2 + off
        nms_scr[b, 0:1, :] = ox1
        nms_scr[b, 1:2, :] = oy1
        nms_scr[b, 2:3, :] = ox2
        nms_scr[b, 3:4, :] = oy2
        nms_scr[b, 4:5, :] = (ox2 - ox1) * (oy2 - oy1)

        def colof(f, selt=selt):
            return jnp.sum(jnp.where(lane == f, selt, 0.0), axis=1,
                           keepdims=True)

        offc = colof(5) * _MAXWH
        ox1c = colof(0) + offc
        oy1c = colof(1) + offc
        ox2c = colof(2) + offc
        oy2c = colof(3) + offc
        areac = (ox2c - ox1c) * (oy2c - oy1c)
        nmst_scr[b] = (jnp.where(lane == 0, ox1c, 0.0)
                       + jnp.where(lane == 1, oy1c, 0.0)
                       + jnp.where(lane == 2, ox2c, 0.0)
                       + jnp.where(lane == 3, oy2c, 0.0)
                       + jnp.where(lane == 4, areac, 0.0))
        keeps.append((sv > _CONF).astype(jnp.float32))

    def nms_step(t, keepT):
        keepT = list(keepT)
        for u in range(4):
            i = t * 4 + u
            for b in range(bs):
                keep = keepT[b]
                rowi = nmst_scr[b, pl.ds(i, 1), :]

                def bx5(f, rowi=rowi):
                    return jnp.sum(jnp.where(lane == f, rowi, 0.0), axis=1,
                                   keepdims=True)

                x1i = bx5(0)
                y1i = bx5(1)
                x2i = bx5(2)
                y2i = bx5(3)
                ai = bx5(4)
                ki = jnp.sum(jnp.where(lane_s == i, keep, 0.0), axis=1,
                             keepdims=True)
                iw = jnp.maximum(
                    jnp.minimum(x2i, nms_scr[b, 2:3, :])
                    - jnp.maximum(x1i, nms_scr[b, 0:1, :]), 0.0)
                ih = jnp.maximum(
                    jnp.minimum(y2i, nms_scr[b, 3:4, :])
                    - jnp.maximum(y1i, nms_scr[b, 1:2, :]), 0.0)
                inter = iw * ih
                iou = inter / (ai + nms_scr[b, 4:5, :] - inter + 1e-9)
                suppress = ((iou > _IOU) & (lane_s > i)
                            & (ki > 0.5)).astype(jnp.float32)
                keepT[b] = keep * (1.0 - suppress)
        return tuple(keepT)

    kfs = jax.lax.fori_loop(0, _MAXDET // 4, nms_step, tuple(keeps))

    for b in range(bs):
        kf = kfs[b]
        out_ref[b, 0:1, :] = sel_scr[b, 0:1, :] * kf
        out_ref[b, 1:2, :] = sel_scr[b, 1:2, :] * kf
        out_ref[b, 2:3, :] = sel_scr[b, 2:3, :] * kf
        out_ref[b, 3:4, :] = sel_scr[b, 3:4, :] * kf
        out_ref[b, 4:5, :] = sel_scr[b, 4:5, :] * kf
        out_ref[b, 5:6, :] = sel_scr[b, 5:6, :] * kf
        out_ref[b, 6:8, :] = jnp.zeros((2, _SELW), jnp.float32)


def kernel(preds, anchors, image_size):
    L, bs, C, H, W = preds.shape
    na = anchors.shape[1]
    nc = C // na
    HW = H * W
    NH = L * na
    N = NH * HW

    img = jnp.asarray(image_size, jnp.float32)
    sh = img / jnp.float32(H)
    sw = img / jnp.float32(W)
    aw = (anchors[..., 0] / sw).reshape(NH)
    ah = (anchors[..., 1] / sh).reshape(NH)
    params = jnp.stack(
        [aw, ah, jnp.broadcast_to(sw, (NH,)), jnp.broadcast_to(sh, (NH,))],
        axis=-1)

    p5 = preds.reshape(L, bs, na, nc, HW)

    decode = pl.pallas_call(
        functools.partial(_decode_body, nc=nc, H=H, W=W),
        grid=(L, bs, na),
        in_specs=[
            pl.BlockSpec(memory_space=pltpu.SMEM),
            pl.BlockSpec((1, 1, 1, nc, HW), lambda l, b, a: (l, b, a, 0, 0)),
        ],
        out_specs=[
            pl.BlockSpec((1, 1, 1, HW), lambda l, b, a: (b, l * na + a, 0, 0))
        ] * 6,
        out_shape=[jax.ShapeDtypeStruct((bs, NH, 1, HW), jnp.float32)] * 6,
    )
    x1, y1, x2, y2, cf, cl = decode(params, p5)

    lw = 128
    rows = N // lw
    def rsh(t):
        return t.reshape(bs, rows, lw)
    x1, y1, x2, y2, cf, cl = map(rsh, (x1, y1, x2, y2, cf, cl))

    nms = pl.pallas_call(
        functools.partial(_nms_body, rows=rows, lw=lw, bs=bs),
        out_shape=jax.ShapeDtypeStruct((bs, 8, _SELW), jnp.float32),
        scratch_shapes=[
            pltpu.VMEM((bs, rows, lw), jnp.float32),
            pltpu.VMEM((bs, _SELW, _SELW), jnp.float32),
            pltpu.VMEM((bs, _SELW, lw), jnp.float32),
            pltpu.VMEM((bs, _SELW, 6 * lw), jnp.float32),
            pltpu.VMEM((bs, 8, _SELW), jnp.float32),
            pltpu.VMEM((bs, 8, _SELW), jnp.float32),
            pltpu.VMEM((bs, _SELW, lw), jnp.float32),
        ],
    )
    det8 = nms(x1, y1, x2, y2, cf, cl)
    return det8[:, :6, :_MAXDET].transpose(0, 2, 1)


# speedup vs baseline: 2.1841x; 1.0064x over previous
import functools

import jax
import jax.numpy as jnp
from jax.experimental import pallas as pl
from jax.experimental.pallas import tpu as pltpu

_CONF = 0.2
_IOU = 0.6
_MAXDET = 300
_MAXWH = 4096.0
_SELW = 384


def _decode_body(params_ref, preds_ref, x1_ref, y1_ref, x2_ref, y2_ref,
                 cf_ref, cl_ref, *, nc, H, W):
    l = pl.program_id(0)
    a = pl.program_id(2)
    na = pl.num_programs(2)
    h = l * na + a
    aw = params_ref[h, 0]
    ah = params_ref[h, 1]
    sw = params_ref[h, 2]
    sh = params_ref[h, 3]

    p = preds_ref[0, 0, 0]
    s = jax.nn.sigmoid(p)

    ni = jax.lax.broadcasted_iota(jnp.int32, (1, H * W), 1)
    xg = (ni % W).astype(jnp.float32)
    yg = (ni // W).astype(jnp.float32)

    xc = (s[0:1, :] * 3.0 - 1.0 + xg) * sw
    yc = (s[1:2, :] * 3.0 - 1.0 + yg) * sh
    w = ((s[2:3, :] * 2.0) ** 2 * aw) * sw
    hh = ((s[3:4, :] * 2.0) ** 2 * ah) * sh

    x1_ref[0, 0] = xc - w / 2.0
    y1_ref[0, 0] = yc - hh / 2.0
    x2_ref[0, 0] = xc + w / 2.0
    y2_ref[0, 0] = yc + hh / 2.0

    obj = s[4:5, :]
    cls_s = s[5:, :] * obj
    conf = jnp.max(cls_s, axis=0, keepdims=True)
    kio = jax.lax.broadcasted_iota(jnp.int32, (nc - 5, H * W), 0)
    cls_i = jnp.min(jnp.where(cls_s == conf, kio, jnp.int32(1 << 30)),
                    axis=0, keepdims=True)
    cf_ref[0, 0] = jnp.where(conf > _CONF, conf, 0.0)
    cl_ref[0, 0] = cls_i.astype(jnp.float32)


def _nms_body(x1_ref, y1_ref, x2_ref, y2_ref, cf_ref, cl_ref, out_ref,
              vals_scr, rh_scr, ch_scr, x6_scr, sel_scr, nms_scr, nmst_scr,
              *, rows, lw, bs):
    pad = _SELW
    refs = (x1_ref, y1_ref, x2_ref, y2_ref, cf_ref, cl_ref)
    for b in range(bs):
        vals_scr[b] = cf_ref[b]
        rh_scr[b] = jnp.zeros((pad, pad), jnp.float32)
        ch_scr[b] = jnp.zeros((pad, lw), jnp.float32)
        for f, ref in enumerate(refs):
            x6_scr[b, 0:rows, f * lw:(f + 1) * lw] = ref[b]
        x6_scr[b, rows:pad, :] = jnp.zeros((pad - rows, 6 * lw), jnp.float32)

    lane = jax.lax.broadcasted_iota(jnp.int32, (1, lw), 1)
    lane_s = jax.lax.broadcasted_iota(jnp.int32, (1, _SELW), 1)
    big = jnp.int32(1 << 30)
    nchunk = rows // 8
    iota8 = (jax.lax.broadcasted_iota(jnp.int32, (8, lw), 0) * lw
             + jax.lax.broadcasted_iota(jnp.int32, (8, lw), 1))
    cspan = 8 * lw

    def sel_body(i, _):
        for b in range(bs):
            chunks = [vals_scr[b, k * 8:(k + 1) * 8, :]
                      for k in range(nchunk)]
            lv = chunks
            while len(lv) > 1:
                nxt = [jnp.maximum(lv[2 * j], lv[2 * j + 1])
                       for j in range(len(lv) // 2)]
                if len(lv) % 2:
                    nxt.append(lv[-1])
                lv = nxt
            m = jnp.max(jnp.max(lv[0], axis=0, keepdims=True), axis=1,
                        keepdims=True)
            pos = [(jnp.where(c == m, iota8, big), 1) for c in chunks]
            while len(pos) > 1:
                nxt = []
                for j in range(len(pos) // 2):
                    (va, la), (vb, lb) = pos[2 * j], pos[2 * j + 1]
                    nxt.append((jnp.minimum(va, vb + la * cspan), la + lb))
                if len(pos) % 2:
                    nxt.append(pos[-1])
                pos = nxt
            nv = jnp.min(jnp.min(pos[0][0], axis=0, keepdims=True), axis=1,
                         keepdims=True)
            for k in range(nchunk):
                lc = nv - k * cspan
                vals_scr[b, k * 8:(k + 1) * 8, :] = jnp.where(
                    iota8 == lc, -1.0, vals_scr[b, k * 8:(k + 1) * 8, :])
            rv = nv // lw
            cv = nv - rv * lw
            rh_scr[b, pl.ds(i, 1), :] = (lane_s == rv).astype(jnp.float32)
            ch_scr[b, pl.ds(i, 1), :] = (lane == cv).astype(jnp.float32)
        return 0

    jax.lax.fori_loop(0, _MAXDET, sel_body, 0, unroll=6)

    ident = (jax.lax.broadcasted_iota(jnp.int32, (lw, lw), 0)
             == jax.lax.broadcasted_iota(jnp.int32, (lw, lw), 1)
             ).astype(jnp.float32)
    keeps = []
    for b in range(bs):
        rowsel6 = jax.lax.dot_general(
            rh_scr[b], x6_scr[b], (((1,), (0,)), ((), ())),
            precision=jax.lax.Precision.HIGHEST,
            preferred_element_type=jnp.float32)
        ch = ch_scr[b]
        selt = jnp.zeros((pad, lw), jnp.float32)
        for f in range(6):
            colv = jnp.sum(rowsel6[:, f * lw:(f + 1) * lw] * ch,
                           axis=1, keepdims=True)
            selt = selt + jnp.where(lane == f, colv, 0.0)
        t = jax.lax.dot_general(ident, selt, (((1,), (1,)), ((), ())),
                                precision=jax.lax.Precision.HIGHEST,
                                preferred_element_type=jnp.float32)
        for f in range(6):
            sel_scr[b, f:f + 1, :] = t[f:f + 1, :]

        sx1 = sel_scr[b, 0:1, :]
        sy1 = sel_scr[b, 1:2, :]
        sx2 = sel_scr[b, 2:3, :]
        sy2 = sel_scr[b, 3:4, :]
        sv = sel_scr[b, 4:5, :]
        scl = sel_scr[b, 5:6, :]

        off = scl * _MAXWH
        ox1 = sx1 + off
        oy1 = sy1 + off
        ox2 = sx2 + off
        oy2 = sy2 + off
        nms_scr[b, 0:1, :] = ox1
        nms_scr[b, 1:2, :] = oy1
        nms_scr[b, 2:3, :] = ox2
        nms_scr[b, 3:4, :] = oy2
        nms_scr[b, 4:5, :] = (ox2 - ox1) * (oy2 - oy1)

        def colof(f, selt=selt):
            return jnp.sum(jnp.where(lane == f, selt, 0.0), axis=1,
                           keepdims=True)

        offc = colof(5) * _MAXWH
        ox1c = colof(0) + offc
        oy1c = colof(1) + offc
        ox2c = colof(2) + offc
        oy2c = colof(3) + offc
        areac = (ox2c - ox1c) * (oy2c - oy1c)
        nmst_scr[b] = (jnp.where(lane == 0, ox1c, 0.0)
                       + jnp.where(lane == 1, oy1c, 0.0)
                       + jnp.where(lane == 2, ox2c, 0.0)
                       + jnp.where(lane == 3, oy2c, 0.0)
                       + jnp.where(lane == 4, areac, 0.0))
        keeps.append((sv > _CONF).astype(jnp.float32))

    def nms_step(t, keepT):
        keepT = list(keepT)
        for u in range(6):
            i = t * 6 + u
            for b in range(bs):
                keep = keepT[b]
                rowi = nmst_scr[b, pl.ds(i, 1), :]

                def bx5(f, rowi=rowi):
                    return jnp.sum(jnp.where(lane == f, rowi, 0.0), axis=1,
                                   keepdims=True)

                x1i = bx5(0)
                y1i = bx5(1)
                x2i = bx5(2)
                y2i = bx5(3)
                ai = bx5(4)
                ki = jnp.sum(jnp.where(lane_s == i, keep, 0.0), axis=1,
                             keepdims=True)
                iw = jnp.maximum(
                    jnp.minimum(x2i, nms_scr[b, 2:3, :])
                    - jnp.maximum(x1i, nms_scr[b, 0:1, :]), 0.0)
                ih = jnp.maximum(
                    jnp.minimum(y2i, nms_scr[b, 3:4, :])
                    - jnp.maximum(y1i, nms_scr[b, 1:2, :]), 0.0)
                inter = iw * ih
                iou = inter / (ai + nms_scr[b, 4:5, :] - inter + 1e-9)
                suppress = ((iou > _IOU) & (lane_s > i)
                            & (ki > 0.5)).astype(jnp.float32)
                keepT[b] = keep * (1.0 - suppress)
        return tuple(keepT)

    kfs = jax.lax.fori_loop(0, _MAXDET // 6, nms_step, tuple(keeps))

    for b in range(bs):
        kf = kfs[b]
        out_ref[b, 0:1, :] = sel_scr[b, 0:1, :] * kf
        out_ref[b, 1:2, :] = sel_scr[b, 1:2, :] * kf
        out_ref[b, 2:3, :] = sel_scr[b, 2:3, :] * kf
        out_ref[b, 3:4, :] = sel_scr[b, 3:4, :] * kf
        out_ref[b, 4:5, :] = sel_scr[b, 4:5, :] * kf
        out_ref[b, 5:6, :] = sel_scr[b, 5:6, :] * kf
        out_ref[b, 6:8, :] = jnp.zeros((2, _SELW), jnp.float32)


def kernel(preds, anchors, image_size):
    L, bs, C, H, W = preds.shape
    na = anchors.shape[1]
    nc = C // na
    HW = H * W
    NH = L * na
    N = NH * HW

    img = jnp.asarray(image_size, jnp.float32)
    sh = img / jnp.float32(H)
    sw = img / jnp.float32(W)
    aw = (anchors[..., 0] / sw).reshape(NH)
    ah = (anchors[..., 1] / sh).reshape(NH)
    params = jnp.stack(
        [aw, ah, jnp.broadcast_to(sw, (NH,)), jnp.broadcast_to(sh, (NH,))],
        axis=-1)

    p5 = preds.reshape(L, bs, na, nc, HW)

    decode = pl.pallas_call(
        functools.partial(_decode_body, nc=nc, H=H, W=W),
        grid=(L, bs, na),
        in_specs=[
            pl.BlockSpec(memory_space=pltpu.SMEM),
            pl.BlockSpec((1, 1, 1, nc, HW), lambda l, b, a: (l, b, a, 0, 0)),
        ],
        out_specs=[
            pl.BlockSpec((1, 1, 1, HW), lambda l, b, a: (b, l * na + a, 0, 0))
        ] * 6,
        out_shape=[jax.ShapeDtypeStruct((bs, NH, 1, HW), jnp.float32)] * 6,
    )
    x1, y1, x2, y2, cf, cl = decode(params, p5)

    lw = 128
    rows = N // lw
    def rsh(t):
        return t.reshape(bs, rows, lw)
    x1, y1, x2, y2, cf, cl = map(rsh, (x1, y1, x2, y2, cf, cl))

    nms = pl.pallas_call(
        functools.partial(_nms_body, rows=rows, lw=lw, bs=bs),
        out_shape=jax.ShapeDtypeStruct((bs, 8, _SELW), jnp.float32),
        scratch_shapes=[
            pltpu.VMEM((bs, rows, lw), jnp.float32),
            pltpu.VMEM((bs, _SELW, _SELW), jnp.float32),
            pltpu.VMEM((bs, _SELW, lw), jnp.float32),
            pltpu.VMEM((bs, _SELW, 6 * lw), jnp.float32),
            pltpu.VMEM((bs, 8, _SELW), jnp.float32),
            pltpu.VMEM((bs, 8, _SELW), jnp.float32),
            pltpu.VMEM((bs, _SELW, lw), jnp.float32),
        ],
    )
    det8 = nms(x1, y1, x2, y2, cf, cl)
    return det8[:, :6, :_MAXDET].transpose(0, 2, 1)
